# Initial kernel scaffold; baseline (speedup 1.0000x reference)
import jax
import jax.numpy as jnp
from jax.experimental import pallas as pl


def kernel(x, edge_index, W_self0, W_neigh0, b0, W_self1, W_neigh1, b1, W_self2, W_neigh2, b2):
    raise NotImplementedError("write your pallas kernel here")


# speedup vs baseline: 12.3125x; 12.3125x over previous
import functools

import jax
import jax.numpy as jnp
from jax import lax
from jax.experimental import pallas as pl
from jax.experimental.pallas import tpu as pltpu
from jax.experimental.pallas import tpu_sc as plsc

N = 10000
E = 320000
D = 128
NC = 2
NS = 16
NW = NC * NS
C = 80
K = 5
EPT = E // NS
G = EPT // C // K
AP = 10240
RPT = AP // NS
ZR = 32
DW = 16


def _make_agg(WH, with_deg):
  mesh = plsc.VectorSubcoreMesh(core_axis_name="c", subcore_axis_name="s",
                                num_cores=NC, num_subcores=NS)
  out_type = [jax.ShapeDtypeStruct((NC, AP, WH), jnp.float32)]
  scratch = (
      [pltpu.VMEM((C,), jnp.int32) for _ in range(K)]
      + [pltpu.VMEM((C,), jnp.int32) for _ in range(K)]
      + [pltpu.VMEM((C, WH), jnp.float32) for _ in range(K)]
      + [
          pltpu.VMEM((EPT,), jnp.int32),
          pltpu.VMEM((EPT,), jnp.int32),
          pltpu.VMEM((ZR, WH), jnp.float32),
          pltpu.VMEM_SHARED((AP, WH), jnp.float32),
          pltpu.SemaphoreType.DMA,
          pltpu.SemaphoreType.DMA,
      ]
  )
  if with_deg:
    out_type.append(jax.ShapeDtypeStruct((NC, AP, DW), jnp.float32))
    scratch += [
        pltpu.VMEM((C, DW), jnp.float32),
        pltpu.VMEM((ZR, DW), jnp.float32),
        pltpu.VMEM_SHARED((AP, DW), jnp.float32),
    ]

  def body(srcv, dstv, feat2, *refs):
    if with_deg:
      out_acc, out_deg, *rest = refs
      (*bufs, src_all, dst_all, zbuf, acc, sem, sem2,
       ones_v, zbufd, dacc) = rest
    else:
      out_acc, *rest = refs
      (*bufs, src_all, dst_all, zbuf, acc, sem, sem2) = rest
    src_idx = bufs[0:K]
    dst_idx = bufs[K:2 * K]
    rows = bufs[2 * K:3 * K]
    cid = lax.axis_index("c")
    sid = lax.axis_index("s")

    lda = pltpu.async_copy(srcv.at[pl.ds(sid * EPT, EPT)], src_all, sem)
    ldb = pltpu.async_copy(dstv.at[pl.ds(sid * EPT, EPT)], dst_all, sem)

    def zb(i, _):
      r = i // (WH // 16)
      c = i % (WH // 16)
      zbuf[r, pl.ds(c * 16, 16)] = jnp.zeros((16,), jnp.float32)
      return 0
    lax.fori_loop(0, ZR * (WH // 16), zb, 0)
    if with_deg:
      def zbd(i, _):
        zbufd[i, pl.ds(0, 16)] = jnp.zeros((16,), jnp.float32)
        return 0
      lax.fori_loop(0, ZR, zbd, 0)
      def ob(i, _):
        ones_v[i, :] = jnp.ones((16,), jnp.float32)
        return 0
      lax.fori_loop(0, C, ob, 0)

    def zc(j, _):
      pltpu.sync_copy(zbuf, acc.at[pl.ds(sid * RPT + j * ZR, ZR)])
      if with_deg:
        pltpu.sync_copy(zbufd, dacc.at[pl.ds(sid * RPT + j * ZR, ZR)])
      return 0
    lax.fori_loop(0, RPT // ZR, zc, 0)
    lda.wait()
    ldb.wait()
    plsc.subcore_barrier()

    def grp(g, _):
      gats = []
      for r in range(K):
        e0 = (g * K + r) * C
        for j in range(C // 16):
          sv = src_all[pl.ds(e0 + j * 16, 16)]
          src_idx[r][pl.ds(j * 16, 16)] = sv * 2 + cid
          dst_idx[r][pl.ds(j * 16, 16)] = dst_all[pl.ds(e0 + j * 16, 16)]
        gats.append(pltpu.async_copy(feat2.at[src_idx[r]], rows[r], sem))
      if with_deg:
        deg_do = lax.select(cid == 0, g < G // 2, g >= G // 2)
      scats = []
      for r in range(K):
        gats[r].wait()
        scats.append(
            pltpu.async_copy(rows[r], acc.at[dst_idx[r]], sem2, add=True))
        if with_deg:
          @pl.when(deg_do)
          def _():
            pltpu.sync_copy(ones_v, dacc.at[dst_idx[r]], add=True)
      for sc in scats:
        sc.wait()
      return 0
    lax.fori_loop(0, G, grp, 0)
    plsc.subcore_barrier()

    pltpu.sync_copy(acc.at[pl.ds(sid * RPT, RPT)],
                    out_acc.at[cid, pl.ds(sid * RPT, RPT)])
    if with_deg:
      pltpu.sync_copy(dacc.at[pl.ds(sid * RPT, RPT)],
                      out_deg.at[cid, pl.ds(sid * RPT, RPT)])

  return pl.kernel(body, out_type=tuple(out_type) if with_deg else out_type[0],
                   mesh=mesh, scratch_types=scratch,
                   compiler_params=pltpu.CompilerParams(
                       use_tc_tiling_on_sc=False))


_make_agg = functools.lru_cache(maxsize=None)(_make_agg)


_R = 1000
_GRID = N // _R


def _tc0_body(x_r, acc_r, dacc_r, ws_r, wn_r, b_r, h1_r, rec_r):
  s = jnp.concatenate([acc_r[0], acc_r[1]], axis=-1)
  d16 = dacc_r[0] + dacc_r[1]
  deg = jnp.sum(d16, axis=1, keepdims=True) * (1.0 / 16.0)
  recip = 1.0 / jnp.maximum(deg, 1.0)
  hn = s * recip
  h1 = (jnp.dot(x_r[...], ws_r[...], preferred_element_type=jnp.float32)
        + jnp.dot(hn, wn_r[...], preferred_element_type=jnp.float32)
        + b_r[...])
  h1_r[...] = jnp.maximum(h1, 0.0)
  rec_r[...] = jnp.broadcast_to(recip, (recip.shape[0], D))


def _tc1_body(h1_r, acc_r, rec_r, ws_r, wn_r, b_r, wn2_r, h2_r, m2_r):
  hn = jnp.concatenate([acc_r[0], acc_r[1]], axis=-1) * rec_r[...]
  h2 = (jnp.dot(h1_r[...], ws_r[...], preferred_element_type=jnp.float32)
        + jnp.dot(hn, wn_r[...], preferred_element_type=jnp.float32)
        + b_r[...])
  h2 = jnp.maximum(h2, 0.0)
  h2_r[...] = h2
  m2_r[...] = jnp.dot(h2, wn2_r[...], preferred_element_type=jnp.float32)


def _tc2_body(h2_r, acc_r, rec_r, ws_r, b_r, out_r):
  hn = jnp.concatenate([acc_r[0], acc_r[1]], axis=-1) * rec_r[:, 0:64]
  out_r[...] = (jnp.dot(h2_r[...], ws_r[...],
                        preferred_element_type=jnp.float32)
                + hn + b_r[...])


def _rows_spec(w):
  return pl.BlockSpec((_R, w), lambda i: (i, 0))


def _acc_spec(w):
  return pl.BlockSpec((NC, _R, w), lambda i: (0, i, 0))


def _whole_spec(a, b):
  return pl.BlockSpec((a, b), lambda i: (0, 0))


_tc0 = pl.pallas_call(
    _tc0_body,
    grid=(_GRID,),
    in_specs=[_rows_spec(D), _acc_spec(D // 2), _acc_spec(DW),
              _whole_spec(D, D), _whole_spec(D, D), _whole_spec(1, D)],
    out_specs=[_rows_spec(D), _rows_spec(D)],
    out_shape=[jax.ShapeDtypeStruct((N, D), jnp.float32),
               jax.ShapeDtypeStruct((N, D), jnp.float32)],
)

_tc1 = pl.pallas_call(
    _tc1_body,
    grid=(_GRID,),
    in_specs=[_rows_spec(D), _acc_spec(D // 2), _rows_spec(D),
              _whole_spec(D, D), _whole_spec(D, D), _whole_spec(1, D),
              _whole_spec(D, 64)],
    out_specs=[_rows_spec(D), _rows_spec(64)],
    out_shape=[jax.ShapeDtypeStruct((N, D), jnp.float32),
               jax.ShapeDtypeStruct((N, 64), jnp.float32)],
)

_tc2 = pl.pallas_call(
    _tc2_body,
    grid=(_GRID,),
    in_specs=[_rows_spec(D), _acc_spec(32), _rows_spec(D),
              _whole_spec(D, 64), _whole_spec(1, 64)],
    out_specs=_rows_spec(64),
    out_shape=jax.ShapeDtypeStruct((N, 64), jnp.float32),
)


def kernel(x, edge_index, W_self0, W_neigh0, b0,
           W_self1, W_neigh1, b1, W_self2, W_neigh2, b2):
  src = edge_index[0]
  dst = edge_index[1]

  acc0, dacc0 = _make_agg(D // 2, True)(src, dst, x.reshape(2 * N, D // 2))
  h1, recb = _tc0(x, acc0, dacc0, W_self0, W_neigh0, b0.reshape(1, D))

  acc1 = _make_agg(D // 2, False)(src, dst, h1.reshape(2 * N, D // 2))
  wn2p = jnp.pad(W_neigh2, ((0, 0), (0, 17)))
  h2, m2 = _tc1(h1, acc1, recb, W_self1, W_neigh1, b1.reshape(1, D), wn2p)

  acc2 = _make_agg(32, False)(src, dst, m2.reshape(2 * N, 32))
  ws2p = jnp.pad(W_self2, ((0, 0), (0, 17)))
  b2p = jnp.pad(b2, (0, 17)).reshape(1, 64)
  outp = _tc2(h2, acc2, recb, ws2p, b2p)
  return outp[:, :47]
